# baseline (speedup 1.0000x reference)
import functools

import jax
import jax.numpy as jnp
from jax import lax
from jax.experimental import pallas as pl
from jax.experimental.pallas import tpu as pltpu
from jax.experimental.pallas import tpu_sc as plsc

NEG_SLOPE = 0.2
OUT = 16
HID = 64
N_DST = 10000
N_SRC = 10000
E_TOTAL = 320000

NC = 2
NS = 16
NW = NC * NS
LANES = 16

HALF = OUT // 2
EPP = E_TOTAL // NS
CHUNK = 400
NCHUNK = EPP // CHUNK
GROUPS = CHUNK // LANES


def _proj_body(xs_ref, ws_ref, bs_ref, atts_ref, xa_ref, wa_ref, ba_ref,
               attd_ref, hs_ref, asrc_ref, adst_ref):
    hs = jnp.dot(xs_ref[...], ws_ref[...],
                 preferred_element_type=jnp.float32) + bs_ref[...]
    hs_ref[...] = hs
    asrc_ref[...] = jnp.dot(hs, atts_ref[...],
                            preferred_element_type=jnp.float32)
    ha = jnp.dot(xa_ref[...], wa_ref[...],
                 preferred_element_type=jnp.float32) + ba_ref[...]
    adst_ref[...] = jnp.dot(ha, attd_ref[...],
                            preferred_element_type=jnp.float32)


def _edge_body(ei_ref, asrc_hbm, adst_hbm, hs_hbm,
               out0_hbm, out1_hbm, deno_hbm,
               asrc_v, adst_v, src0_v, src1_v, dst0_v, dst1_v, hr0_v, hr1_v,
               accum_v, den_v, semi_s, semi_d, semh):
    cid = lax.axis_index("c")
    sid = lax.axis_index("s")
    hoff = cid * HALF

    src_b = [src0_v, src1_v]
    dst_b = [dst0_v, dst1_v]
    hr_b = [hr0_v, hr1_v]

    pltpu.sync_copy(asrc_hbm, asrc_v)
    pltpu.sync_copy(adst_hbm, adst_v)

    zeros16 = jnp.zeros((LANES,), jnp.float32)

    def za(r, carry):
        accum_v[pl.ds(r * LANES, LANES)] = zeros16
        return carry
    lax.fori_loop(0, N_DST * HALF // LANES, za, 0, unroll=8)

    def zd(r, carry):
        den_v[pl.ds(r * LANES, LANES)] = zeros16
        return carry
    lax.fori_loop(0, N_DST // LANES, zd, 0, unroll=8)

    iota = lax.iota(jnp.int32, LANES)

    def issue_idx(c, b):
        off = sid * EPP + c * CHUNK
        d1 = pltpu.async_copy(ei_ref.at[pl.ds(off, CHUNK)], src_b[b], semi_s)
        d2 = pltpu.async_copy(ei_ref.at[pl.ds(E_TOTAL + off, CHUNK)],
                              dst_b[b], semi_d)
        return (d1, d2)

    def issue_gather(b):
        return pltpu.async_copy(hs_hbm.at[src_b[b]], hr_b[b], semh)

    def compute_chunk(b):
        src_v, dst_v, hrows_v = src_b[b], dst_b[b], hr_b[b]

        def group_body(g, gcarry):
            s16 = src_v[pl.ds(g * LANES, LANES)]
            d16 = dst_v[pl.ds(g * LANES, LANES)]
            a = plsc.load_gather(asrc_v, [s16]) + plsc.load_gather(adst_v, [d16])
            a = jnp.where(a >= 0.0, a, a * NEG_SLOPE)
            w = jnp.exp(a)
            rows = g * LANES + iota
            for j in range(HALF):
                jv = jnp.full((LANES,), j, jnp.int32) + hoff
                col = plsc.load_gather(hrows_v, [rows, jv])
                plsc.addupdate_scatter(accum_v, [d16 + j * N_DST], col * w)
            plsc.addupdate_scatter(den_v, [d16], w)
            return gcarry
        lax.fori_loop(0, GROUPS, group_body, 0)

    idx_d = issue_idx(0, 0)
    idx_d[0].wait()
    idx_d[1].wait()
    gat_d = issue_gather(0)
    for c in range(NCHUNK):
        b = c % 2
        nb = 1 - b
        if c + 1 < NCHUNK:
            idx_d = issue_idx(c + 1, nb)
        gat_d.wait()
        if c + 1 < NCHUNK:
            idx_d[0].wait()
            idx_d[1].wait()
            gat_d = issue_gather(nb)
        compute_chunk(b)

    @pl.when(cid == 0)
    def _copy0():
        pltpu.sync_copy(accum_v, out0_hbm.at[sid])
        pltpu.sync_copy(den_v, deno_hbm.at[sid])

    @pl.when(cid == 1)
    def _copy1():
        pltpu.sync_copy(accum_v, out1_hbm.at[sid])


def _tail_body(p0_ref, p1_ref, den_ref, w1t_ref, b1_ref, w2t_ref, b2_ref,
               out_ref):
    n0 = jnp.sum(p0_ref[...], axis=0)
    n1 = jnp.sum(p1_ref[...], axis=0)
    num = jnp.concatenate([n0, n1], axis=0)
    den = jnp.sum(den_ref[...], axis=0, keepdims=True)
    outa = jnp.maximum(num / (den + 1e-16), 0.0)
    h1 = jnp.maximum(jnp.dot(w1t_ref[...], outa,
                             preferred_element_type=jnp.float32) + b1_ref[...], 0.0)
    logits = jnp.dot(w2t_ref[...], h1,
                     preferred_element_type=jnp.float32) + b2_ref[...]
    m = jnp.max(logits)
    e = jnp.exp(logits - m)
    out_ref[...] = e / jnp.sum(e)


@jax.jit
def kernel(x_a, x_s, edge_index_sa, edge_index_as, W_a, b_a, W_s, b_s,
           att_src_sa, att_dst_sa, att_src_as, att_dst_as, q, Wk, bk,
           W1, b1, W2, b2):
    del edge_index_as, att_src_as, att_dst_as, q, Wk, bk

    hs, asrc2, adst2 = pl.pallas_call(
        _proj_body,
        out_shape=[
            jax.ShapeDtypeStruct((N_SRC, OUT), jnp.float32),
            jax.ShapeDtypeStruct((N_SRC, 1), jnp.float32),
            jax.ShapeDtypeStruct((N_DST, 1), jnp.float32),
        ],
    )(x_s, W_s, b_s.reshape(1, OUT), att_src_sa.reshape(OUT, 1),
      x_a, W_a, b_a.reshape(1, OUT), att_dst_sa.reshape(OUT, 1))

    mesh = plsc.VectorSubcoreMesh(core_axis_name="c", subcore_axis_name="s",
                                  num_cores=NC, num_subcores=NS)
    edge_call = pl.kernel(
        _edge_body,
        out_type=[
            jax.ShapeDtypeStruct((NS, N_DST * HALF), jnp.float32),
            jax.ShapeDtypeStruct((NS, N_DST * HALF), jnp.float32),
            jax.ShapeDtypeStruct((NS, N_DST), jnp.float32),
        ],
        mesh=mesh,
        compiler_params=pltpu.CompilerParams(needs_layout_passes=False,
                                             use_tc_tiling_on_sc=False),
        scratch_types=[
            pltpu.VMEM((N_SRC,), jnp.float32),
            pltpu.VMEM((N_DST,), jnp.float32),
            pltpu.VMEM((CHUNK,), jnp.int32),
            pltpu.VMEM((CHUNK,), jnp.int32),
            pltpu.VMEM((CHUNK,), jnp.int32),
            pltpu.VMEM((CHUNK,), jnp.int32),
            pltpu.VMEM((CHUNK, OUT), jnp.float32),
            pltpu.VMEM((CHUNK, OUT), jnp.float32),
            pltpu.VMEM((N_DST * HALF,), jnp.float32),
            pltpu.VMEM((N_DST,), jnp.float32),
            pltpu.SemaphoreType.DMA,
            pltpu.SemaphoreType.DMA,
            pltpu.SemaphoreType.DMA,
        ],
    )
    part0, part1, dens = edge_call(edge_index_sa.reshape(2 * E_TOTAL),
                                   asrc2.reshape(N_SRC), adst2.reshape(N_DST),
                                   hs)

    out = pl.pallas_call(
        _tail_body,
        out_shape=jax.ShapeDtypeStruct((1, N_DST), jnp.float32),
    )(part0.reshape(NS, HALF, N_DST), part1.reshape(NS, HALF, N_DST), dens,
      W1.T, b1.reshape(HID, 1), W2.T, b2.reshape(1, 1))
    return out.reshape(N_DST, 1)


# speedup vs baseline: 1.0999x; 1.0999x over previous
import functools

import jax
import jax.numpy as jnp
from jax import lax
from jax.experimental import pallas as pl
from jax.experimental.pallas import tpu as pltpu
from jax.experimental.pallas import tpu_sc as plsc

NEG_SLOPE = 0.2
OUT = 16
HID = 64
N_DST = 10000
N_SRC = 10000
E_TOTAL = 320000

NC = 2
NS = 16
NW = NC * NS
LANES = 16

EPW = E_TOTAL // NW
CHUNK = 400
NCHUNK = EPW // CHUNK
GROUPS = CHUNK // LANES
ROWS_PT = 624
ROWS_TAIL = N_DST - NS * ROWS_PT


def _proj_body(xs_ref, ws_ref, bs_ref, atts_ref, xa_ref, wa_ref, ba_ref,
               attd_ref, hs_ref, asrc_ref, adst_ref):
    hs = jnp.dot(xs_ref[...], ws_ref[...],
                 preferred_element_type=jnp.float32) + bs_ref[...]
    hs_ref[...] = hs
    asrc_ref[...] = jnp.dot(hs, atts_ref[...],
                            preferred_element_type=jnp.float32)
    ha = jnp.dot(xa_ref[...], wa_ref[...],
                 preferred_element_type=jnp.float32) + ba_ref[...]
    adst_ref[...] = jnp.dot(ha, attd_ref[...],
                            preferred_element_type=jnp.float32)


def _edge_body(ei_ref, asrc_hbm, adst_hbm, hs_hbm, out_hbm, den_hbm,
               asrc_v, adst_v, src0_v, src1_v, dst0_v, dst1_v,
               hr0_v, hr1_v, msg0_v, msg1_v, den_v,
               accum_sh, semi_s, semi_d, semh, sems_m):
    cid = lax.axis_index("c")
    sid = lax.axis_index("s")
    wid = sid * NC + cid

    src_b = [src0_v, src1_v]
    dst_b = [dst0_v, dst1_v]
    hr_b = [hr0_v, hr1_v]
    msg_b = [msg0_v, msg1_v]

    pltpu.sync_copy(asrc_hbm, asrc_v)
    pltpu.sync_copy(adst_hbm, adst_v)

    zeros16 = jnp.zeros((LANES,), jnp.float32)

    def zrow(r, carry):
        msg0_v[r, pl.ds(0, LANES)] = zeros16
        return carry
    lax.fori_loop(0, CHUNK, zrow, 0)

    def zd(r, carry):
        den_v[pl.ds(r * LANES, LANES)] = zeros16
        return carry
    lax.fori_loop(0, N_DST // LANES, zd, 0, unroll=8)

    base_row = sid * ROWS_PT
    pltpu.sync_copy(msg0_v, accum_sh.at[pl.ds(base_row, CHUNK)])
    pltpu.sync_copy(msg0_v.at[pl.ds(0, ROWS_PT - CHUNK)],
                    accum_sh.at[pl.ds(base_row + CHUNK, ROWS_PT - CHUNK)])

    @pl.when(sid == NS - 1)
    def _init_tail():
        pltpu.sync_copy(msg0_v.at[pl.ds(0, ROWS_TAIL)],
                        accum_sh.at[pl.ds(NS * ROWS_PT, ROWS_TAIL)])
    plsc.subcore_barrier()

    iota = lax.iota(jnp.int32, LANES)

    def issue_idx(c, b):
        off = wid * EPW + c * CHUNK
        d1 = pltpu.async_copy(ei_ref.at[pl.ds(off, CHUNK)], src_b[b], semi_s)
        d2 = pltpu.async_copy(ei_ref.at[pl.ds(E_TOTAL + off, CHUNK)],
                              dst_b[b], semi_d)
        return (d1, d2)

    def issue_gather(b):
        return pltpu.async_copy(hs_hbm.at[src_b[b]], hr_b[b], semh)

    def compute_chunk(b):
        src_v, dst_v, hrows_v, msg_v = src_b[b], dst_b[b], hr_b[b], msg_b[b]

        def group_body(g, gcarry):
            s16 = src_v[pl.ds(g * LANES, LANES)]
            d16 = dst_v[pl.ds(g * LANES, LANES)]
            a = plsc.load_gather(asrc_v, [s16]) + plsc.load_gather(adst_v, [d16])
            a = jnp.where(a >= 0.0, a, a * NEG_SLOPE)
            w = jnp.exp(a)
            plsc.addupdate_scatter(den_v, [d16], w)
            rows = g * LANES + iota
            for j in range(OUT):
                jv = jnp.full((LANES,), j, jnp.int32)
                col = plsc.load_gather(hrows_v, [rows, jv])
                plsc.store_scatter(msg_v, [rows, jv], col * w)
            return gcarry
        lax.fori_loop(0, GROUPS, group_body, 0)

    def issue_scatter(b):
        d1 = pltpu.async_copy(msg_b[b], accum_sh.at[dst_b[b]], sems_m, add=True)
        return (d1,)

    idx_d = issue_idx(0, 0)
    idx_d[0].wait()
    idx_d[1].wait()
    gat_d = issue_gather(0)
    scat_d = None
    for c in range(NCHUNK):
        b = c % 2
        nb = 1 - b
        if scat_d is not None:
            scat_d[0].wait()
        if c + 1 < NCHUNK:
            idx_d = issue_idx(c + 1, nb)
        gat_d.wait()
        if c + 1 < NCHUNK:
            idx_d[0].wait()
            idx_d[1].wait()
            gat_d = issue_gather(nb)
        compute_chunk(b)
        scat_d = issue_scatter(b)
    scat_d[0].wait()

    pltpu.sync_copy(den_v, den_hbm.at[wid])

    plsc.subcore_barrier()
    pltpu.sync_copy(accum_sh.at[pl.ds(base_row, ROWS_PT)],
                    out_hbm.at[cid, pl.ds(base_row, ROWS_PT)])

    @pl.when(sid == NS - 1)
    def _copy_tail():
        pltpu.sync_copy(accum_sh.at[pl.ds(NS * ROWS_PT, ROWS_TAIL)],
                        out_hbm.at[cid, pl.ds(NS * ROWS_PT, ROWS_TAIL)])


def _tail_body(part_ref, den_ref, w1_ref, b1_ref, w2_ref, b2_ref, out_ref):
    num = part_ref[0] + part_ref[1]
    den = jnp.sum(den_ref[...], axis=0).reshape(N_DST, 1)
    outa = jnp.maximum(num / (den + 1e-16), 0.0)
    h1 = jnp.maximum(jnp.dot(outa, w1_ref[...],
                             preferred_element_type=jnp.float32) + b1_ref[...], 0.0)
    logits = jnp.dot(h1, w2_ref[...],
                     preferred_element_type=jnp.float32) + b2_ref[...]
    m = jnp.max(logits)
    e = jnp.exp(logits - m)
    out_ref[...] = e / jnp.sum(e)


@jax.jit
def kernel(x_a, x_s, edge_index_sa, edge_index_as, W_a, b_a, W_s, b_s,
           att_src_sa, att_dst_sa, att_src_as, att_dst_as, q, Wk, bk,
           W1, b1, W2, b2):
    del edge_index_as, att_src_as, att_dst_as, q, Wk, bk

    hs, asrc2, adst2 = pl.pallas_call(
        _proj_body,
        out_shape=[
            jax.ShapeDtypeStruct((N_SRC, OUT), jnp.float32),
            jax.ShapeDtypeStruct((N_SRC, 1), jnp.float32),
            jax.ShapeDtypeStruct((N_DST, 1), jnp.float32),
        ],
    )(x_s, W_s, b_s.reshape(1, OUT), att_src_sa.reshape(OUT, 1),
      x_a, W_a, b_a.reshape(1, OUT), att_dst_sa.reshape(OUT, 1))

    mesh = plsc.VectorSubcoreMesh(core_axis_name="c", subcore_axis_name="s",
                                  num_cores=NC, num_subcores=NS)
    edge_call = pl.kernel(
        _edge_body,
        out_type=[
            jax.ShapeDtypeStruct((NC, N_DST, OUT), jnp.float32),
            jax.ShapeDtypeStruct((NW, N_DST), jnp.float32),
        ],
        mesh=mesh,
        compiler_params=pltpu.CompilerParams(needs_layout_passes=False,
                                             use_tc_tiling_on_sc=False),
        scratch_types=[
            pltpu.VMEM((N_SRC,), jnp.float32),
            pltpu.VMEM((N_DST,), jnp.float32),
            pltpu.VMEM((CHUNK,), jnp.int32),
            pltpu.VMEM((CHUNK,), jnp.int32),
            pltpu.VMEM((CHUNK,), jnp.int32),
            pltpu.VMEM((CHUNK,), jnp.int32),
            pltpu.VMEM((CHUNK, OUT), jnp.float32),
            pltpu.VMEM((CHUNK, OUT), jnp.float32),
            pltpu.VMEM((CHUNK, OUT), jnp.float32),
            pltpu.VMEM((CHUNK, OUT), jnp.float32),
            pltpu.VMEM((N_DST,), jnp.float32),
            pltpu.VMEM_SHARED((N_DST, OUT), jnp.float32),
            pltpu.SemaphoreType.DMA,
            pltpu.SemaphoreType.DMA,
            pltpu.SemaphoreType.DMA,
            pltpu.SemaphoreType.DMA,
        ],
    )
    partial, dens = edge_call(edge_index_sa.reshape(2 * E_TOTAL),
                              asrc2.reshape(N_SRC), adst2.reshape(N_DST), hs)

    out = pl.pallas_call(
        _tail_body,
        out_shape=jax.ShapeDtypeStruct((N_DST, 1), jnp.float32),
    )(partial, dens, W1, b1.reshape(1, HID), W2, b2.reshape(1, 1))
    return out


# speedup vs baseline: 1.3517x; 1.2289x over previous
import functools

import jax
import jax.numpy as jnp
from jax import lax
from jax.experimental import pallas as pl
from jax.experimental.pallas import tpu as pltpu
from jax.experimental.pallas import tpu_sc as plsc

NEG_SLOPE = 0.2
OUT = 16
HID = 64
N_DST = 10000
N_SRC = 10000
E_TOTAL = 320000

NC = 2
NS = 16
NW = NC * NS
LANES = 16

EPW = E_TOTAL // NW
CHUNK = 400
NCHUNK = EPW // CHUNK
GROUPS = CHUNK // LANES
ROWS_PT = 624
ROWS_TAIL = N_DST - NS * ROWS_PT


def _proj_body(xs_ref, ws_ref, bs_ref, atts_ref, xa_ref, wa_ref, ba_ref,
               attd_ref, hs_ref, asrc_ref, adst_ref):
    hs = jnp.dot(xs_ref[...], ws_ref[...],
                 preferred_element_type=jnp.float32) + bs_ref[...]
    hs_ref[...] = hs
    asrc_ref[...] = jnp.dot(hs, atts_ref[...],
                            preferred_element_type=jnp.float32)
    ha = jnp.dot(xa_ref[...], wa_ref[...],
                 preferred_element_type=jnp.float32) + ba_ref[...]
    adst_ref[...] = jnp.dot(ha, attd_ref[...],
                            preferred_element_type=jnp.float32)


def _edge_body(ei_ref, asrc_hbm, adst_hbm, hs_hbm, out_hbm, den_hbm,
               asrc_v, adst_v, src0_v, src1_v, dst0_v, dst1_v,
               hr0_v, hr1_v, msg0_v, msg1_v, den_v,
               accum_sh, semi_s, semi_d, semh, sems_m):
    cid = lax.axis_index("c")
    sid = lax.axis_index("s")
    wid = sid * NC + cid

    src_b = [src0_v, src1_v]
    dst_b = [dst0_v, dst1_v]
    hr_b = [hr0_v, hr1_v]
    msg_b = [msg0_v, msg1_v]

    pltpu.sync_copy(asrc_hbm, asrc_v)
    pltpu.sync_copy(adst_hbm, adst_v)

    zeros16 = jnp.zeros((LANES,), jnp.float32)

    def zrow(r, carry):
        msg0_v[r, pl.ds(0, LANES)] = zeros16
        return carry
    lax.fori_loop(0, CHUNK, zrow, 0)

    def zd(r, carry):
        den_v[pl.ds(r * LANES, LANES)] = zeros16
        return carry
    lax.fori_loop(0, N_DST // LANES, zd, 0, unroll=8)

    base_row = sid * ROWS_PT
    pltpu.sync_copy(msg0_v, accum_sh.at[pl.ds(base_row, CHUNK)])
    pltpu.sync_copy(msg0_v.at[pl.ds(0, ROWS_PT - CHUNK)],
                    accum_sh.at[pl.ds(base_row + CHUNK, ROWS_PT - CHUNK)])

    @pl.when(sid == NS - 1)
    def _init_tail():
        pltpu.sync_copy(msg0_v.at[pl.ds(0, ROWS_TAIL)],
                        accum_sh.at[pl.ds(NS * ROWS_PT, ROWS_TAIL)])
    plsc.subcore_barrier()

    iota = lax.iota(jnp.int32, LANES)

    def issue_idx(c, b):
        off = wid * EPW + c * CHUNK
        d1 = pltpu.async_copy(ei_ref.at[pl.ds(off, CHUNK)], src_b[b], semi_s)
        d2 = pltpu.async_copy(ei_ref.at[pl.ds(E_TOTAL + off, CHUNK)],
                              dst_b[b], semi_d)
        return (d1, d2)

    def issue_gather(b):
        return pltpu.async_copy(hs_hbm.at[src_b[b]], hr_b[b], semh)

    def compute_chunk(b):
        src_v, dst_v, hrows_v, msg_v = src_b[b], dst_b[b], hr_b[b], msg_b[b]

        def group_body(g, gcarry):
            s16 = src_v[pl.ds(g * LANES, LANES)]
            d16 = dst_v[pl.ds(g * LANES, LANES)]
            a = plsc.load_gather(asrc_v, [s16]) + plsc.load_gather(adst_v, [d16])
            a = jnp.where(a >= 0.0, a, a * NEG_SLOPE)
            w = jnp.exp(a)
            plsc.addupdate_scatter(den_v, [d16], w)
            base = g * LANES
            for e in range(LANES):
                msg_v[base + e, pl.ds(0, LANES)] = (
                    hrows_v[base + e, pl.ds(0, LANES)] * w[e])
            return gcarry
        lax.fori_loop(0, GROUPS, group_body, 0)

    def issue_scatter(b):
        d1 = pltpu.async_copy(msg_b[b], accum_sh.at[dst_b[b]], sems_m, add=True)
        return (d1,)

    idx_d = issue_idx(0, 0)
    idx_d[0].wait()
    idx_d[1].wait()
    gat_d = issue_gather(0)
    scat_d = None
    for c in range(NCHUNK):
        b = c % 2
        nb = 1 - b
        if scat_d is not None:
            scat_d[0].wait()
        if c + 1 < NCHUNK:
            idx_d = issue_idx(c + 1, nb)
        gat_d.wait()
        if c + 1 < NCHUNK:
            idx_d[0].wait()
            idx_d[1].wait()
            gat_d = issue_gather(nb)
        compute_chunk(b)
        scat_d = issue_scatter(b)
    scat_d[0].wait()

    pltpu.sync_copy(den_v, den_hbm.at[wid])

    plsc.subcore_barrier()
    pltpu.sync_copy(accum_sh.at[pl.ds(base_row, ROWS_PT)],
                    out_hbm.at[cid, pl.ds(base_row, ROWS_PT)])

    @pl.when(sid == NS - 1)
    def _copy_tail():
        pltpu.sync_copy(accum_sh.at[pl.ds(NS * ROWS_PT, ROWS_TAIL)],
                        out_hbm.at[cid, pl.ds(NS * ROWS_PT, ROWS_TAIL)])


def _tail_body(part_ref, den_ref, w1_ref, b1_ref, w2_ref, b2_ref, out_ref):
    num = part_ref[0] + part_ref[1]
    den = jnp.sum(den_ref[...], axis=0).reshape(N_DST, 1)
    outa = jnp.maximum(num / (den + 1e-16), 0.0)
    h1 = jnp.maximum(jnp.dot(outa, w1_ref[...],
                             preferred_element_type=jnp.float32) + b1_ref[...], 0.0)
    logits = jnp.dot(h1, w2_ref[...],
                     preferred_element_type=jnp.float32) + b2_ref[...]
    m = jnp.max(logits)
    e = jnp.exp(logits - m)
    out_ref[...] = e / jnp.sum(e)


@jax.jit
def kernel(x_a, x_s, edge_index_sa, edge_index_as, W_a, b_a, W_s, b_s,
           att_src_sa, att_dst_sa, att_src_as, att_dst_as, q, Wk, bk,
           W1, b1, W2, b2):
    del edge_index_as, att_src_as, att_dst_as, q, Wk, bk

    hs, asrc2, adst2 = pl.pallas_call(
        _proj_body,
        out_shape=[
            jax.ShapeDtypeStruct((N_SRC, OUT), jnp.float32),
            jax.ShapeDtypeStruct((N_SRC, 1), jnp.float32),
            jax.ShapeDtypeStruct((N_DST, 1), jnp.float32),
        ],
    )(x_s, W_s, b_s.reshape(1, OUT), att_src_sa.reshape(OUT, 1),
      x_a, W_a, b_a.reshape(1, OUT), att_dst_sa.reshape(OUT, 1))

    mesh = plsc.VectorSubcoreMesh(core_axis_name="c", subcore_axis_name="s",
                                  num_cores=NC, num_subcores=NS)
    edge_call = pl.kernel(
        _edge_body,
        out_type=[
            jax.ShapeDtypeStruct((NC, N_DST, OUT), jnp.float32),
            jax.ShapeDtypeStruct((NW, N_DST), jnp.float32),
        ],
        mesh=mesh,
        compiler_params=pltpu.CompilerParams(needs_layout_passes=False,
                                             use_tc_tiling_on_sc=False),
        scratch_types=[
            pltpu.VMEM((N_SRC,), jnp.float32),
            pltpu.VMEM((N_DST,), jnp.float32),
            pltpu.VMEM((CHUNK,), jnp.int32),
            pltpu.VMEM((CHUNK,), jnp.int32),
            pltpu.VMEM((CHUNK,), jnp.int32),
            pltpu.VMEM((CHUNK,), jnp.int32),
            pltpu.VMEM((CHUNK, OUT), jnp.float32),
            pltpu.VMEM((CHUNK, OUT), jnp.float32),
            pltpu.VMEM((CHUNK, OUT), jnp.float32),
            pltpu.VMEM((CHUNK, OUT), jnp.float32),
            pltpu.VMEM((N_DST,), jnp.float32),
            pltpu.VMEM_SHARED((N_DST, OUT), jnp.float32),
            pltpu.SemaphoreType.DMA,
            pltpu.SemaphoreType.DMA,
            pltpu.SemaphoreType.DMA,
            pltpu.SemaphoreType.DMA,
        ],
    )
    partial, dens = edge_call(edge_index_sa.reshape(2 * E_TOTAL),
                              asrc2.reshape(N_SRC), adst2.reshape(N_DST), hs)

    out = pl.pallas_call(
        _tail_body,
        out_shape=jax.ShapeDtypeStruct((N_DST, 1), jnp.float32),
    )(partial, dens, W1, b1.reshape(1, HID), W2, b2.reshape(1, 1))
    return out


# speedup vs baseline: 1.3548x; 1.0023x over previous
import functools

import jax
import jax.numpy as jnp
from jax import lax
from jax.experimental import pallas as pl
from jax.experimental.pallas import tpu as pltpu
from jax.experimental.pallas import tpu_sc as plsc

NEG_SLOPE = 0.2
OUT = 16
HID = 64
N_DST = 10000
N_SRC = 10000
E_TOTAL = 320000

NC = 2
NS = 16
NW = NC * NS
LANES = 16

EPW = E_TOTAL // NW
CHUNK = 400
NCHUNK = EPW // CHUNK
GROUPS = CHUNK // LANES
ROWS_PT = 624
ROWS_TAIL = N_DST - NS * ROWS_PT


def _proj_body(xs_ref, ws_ref, bs_ref, atts_ref, xa_ref, wa_ref, ba_ref,
               attd_ref, hs_ref, asrc_ref, adst_ref):
    hs = jnp.dot(xs_ref[...], ws_ref[...],
                 preferred_element_type=jnp.float32) + bs_ref[...]
    hs_ref[...] = hs
    asrc_ref[...] = jnp.dot(hs, atts_ref[...],
                            preferred_element_type=jnp.float32)
    ha = jnp.dot(xa_ref[...], wa_ref[...],
                 preferred_element_type=jnp.float32) + ba_ref[...]
    adst_ref[...] = jnp.dot(ha, attd_ref[...],
                            preferred_element_type=jnp.float32)


def _edge_body(ei_ref, asrc_hbm, adst_hbm, hs_hbm, out_hbm, den_hbm,
               asrc_v, adst_v, src0_v, src1_v, dst0_v, dst1_v,
               hr0_v, hr1_v, msg0_v, msg1_v, den_v,
               accum_sh, semi_s, semi_d, semh, sems_m):
    cid = lax.axis_index("c")
    sid = lax.axis_index("s")
    wid = sid * NC + cid

    src_b = [src0_v, src1_v]
    dst_b = [dst0_v, dst1_v]
    hr_b = [hr0_v, hr1_v]
    msg_b = [msg0_v, msg1_v]

    pltpu.sync_copy(asrc_hbm, asrc_v)
    pltpu.sync_copy(adst_hbm, adst_v)

    zeros16 = jnp.zeros((LANES,), jnp.float32)

    def zrow(r, carry):
        msg0_v[r, pl.ds(0, LANES)] = zeros16
        return carry
    lax.fori_loop(0, CHUNK, zrow, 0)

    def zd(r, carry):
        den_v[pl.ds(r * LANES, LANES)] = zeros16
        return carry
    lax.fori_loop(0, N_DST // LANES, zd, 0, unroll=8)

    base_row = sid * ROWS_PT
    pltpu.sync_copy(msg0_v, accum_sh.at[pl.ds(base_row, CHUNK)])
    pltpu.sync_copy(msg0_v.at[pl.ds(0, ROWS_PT - CHUNK)],
                    accum_sh.at[pl.ds(base_row + CHUNK, ROWS_PT - CHUNK)])

    @pl.when(sid == NS - 1)
    def _init_tail():
        pltpu.sync_copy(msg0_v.at[pl.ds(0, ROWS_TAIL)],
                        accum_sh.at[pl.ds(NS * ROWS_PT, ROWS_TAIL)])
    plsc.subcore_barrier()

    iota = lax.iota(jnp.int32, LANES)

    def issue_idx(c, b):
        off = wid * EPW + c * CHUNK
        d1 = pltpu.async_copy(ei_ref.at[pl.ds(off, CHUNK)], src_b[b], semi_s)
        d2 = pltpu.async_copy(ei_ref.at[pl.ds(E_TOTAL + off, CHUNK)],
                              dst_b[b], semi_d)
        return (d1, d2)

    def issue_gather(b):
        return pltpu.async_copy(hs_hbm.at[src_b[b]], hr_b[b], semh)

    def compute_chunk(b):
        src_v, dst_v, hrows_v, msg_v = src_b[b], dst_b[b], hr_b[b], msg_b[b]

        def group_body(g, gcarry):
            s16 = src_v[pl.ds(g * LANES, LANES)]
            d16 = dst_v[pl.ds(g * LANES, LANES)]
            a = plsc.load_gather(asrc_v, [s16]) + plsc.load_gather(adst_v, [d16])
            w = jnp.exp(jnp.maximum(a, a * NEG_SLOPE))
            plsc.addupdate_scatter(den_v, [d16], w)
            base = g * LANES
            for e in range(LANES):
                msg_v[base + e, pl.ds(0, LANES)] = (
                    hrows_v[base + e, pl.ds(0, LANES)] * w[e])
            return gcarry
        lax.fori_loop(0, GROUPS, group_body, 0)

    def issue_scatter(b):
        d1 = pltpu.async_copy(msg_b[b], accum_sh.at[dst_b[b]], sems_m, add=True)
        return (d1,)

    idx_d = issue_idx(0, 0)
    idx_d[0].wait()
    idx_d[1].wait()
    gat_d = issue_gather(0)
    scat_d = None
    for c in range(NCHUNK):
        b = c % 2
        nb = 1 - b
        if scat_d is not None:
            scat_d[0].wait()
        if c + 1 < NCHUNK:
            idx_d = issue_idx(c + 1, nb)
        gat_d.wait()
        if c + 1 < NCHUNK:
            idx_d[0].wait()
            idx_d[1].wait()
            gat_d = issue_gather(nb)
        compute_chunk(b)
        scat_d = issue_scatter(b)
    scat_d[0].wait()

    pltpu.sync_copy(den_v, den_hbm.at[wid])

    plsc.subcore_barrier()
    pltpu.sync_copy(accum_sh.at[pl.ds(base_row, ROWS_PT)],
                    out_hbm.at[cid, pl.ds(base_row, ROWS_PT)])

    @pl.when(sid == NS - 1)
    def _copy_tail():
        pltpu.sync_copy(accum_sh.at[pl.ds(NS * ROWS_PT, ROWS_TAIL)],
                        out_hbm.at[cid, pl.ds(NS * ROWS_PT, ROWS_TAIL)])


def _tail_body(part_ref, den_ref, w1_ref, b1_ref, w2_ref, b2_ref, out_ref):
    num = part_ref[0] + part_ref[1]
    den = jnp.sum(den_ref[...], axis=0).reshape(N_DST, 1)
    outa = jnp.maximum(num / (den + 1e-16), 0.0)
    h1 = jnp.maximum(jnp.dot(outa, w1_ref[...],
                             preferred_element_type=jnp.float32) + b1_ref[...], 0.0)
    logits = jnp.dot(h1, w2_ref[...],
                     preferred_element_type=jnp.float32) + b2_ref[...]
    m = jnp.max(logits)
    e = jnp.exp(logits - m)
    out_ref[...] = e / jnp.sum(e)


@jax.jit
def kernel(x_a, x_s, edge_index_sa, edge_index_as, W_a, b_a, W_s, b_s,
           att_src_sa, att_dst_sa, att_src_as, att_dst_as, q, Wk, bk,
           W1, b1, W2, b2):
    del edge_index_as, att_src_as, att_dst_as, q, Wk, bk

    hs, asrc2, adst2 = pl.pallas_call(
        _proj_body,
        out_shape=[
            jax.ShapeDtypeStruct((N_SRC, OUT), jnp.float32),
            jax.ShapeDtypeStruct((N_SRC, 1), jnp.float32),
            jax.ShapeDtypeStruct((N_DST, 1), jnp.float32),
        ],
    )(x_s, W_s, b_s.reshape(1, OUT), att_src_sa.reshape(OUT, 1),
      x_a, W_a, b_a.reshape(1, OUT), att_dst_sa.reshape(OUT, 1))

    mesh = plsc.VectorSubcoreMesh(core_axis_name="c", subcore_axis_name="s",
                                  num_cores=NC, num_subcores=NS)
    edge_call = pl.kernel(
        _edge_body,
        out_type=[
            jax.ShapeDtypeStruct((NC, N_DST, OUT), jnp.float32),
            jax.ShapeDtypeStruct((NW, N_DST), jnp.float32),
        ],
        mesh=mesh,
        compiler_params=pltpu.CompilerParams(needs_layout_passes=False,
                                             use_tc_tiling_on_sc=False),
        scratch_types=[
            pltpu.VMEM((N_SRC,), jnp.float32),
            pltpu.VMEM((N_DST,), jnp.float32),
            pltpu.VMEM((CHUNK,), jnp.int32),
            pltpu.VMEM((CHUNK,), jnp.int32),
            pltpu.VMEM((CHUNK,), jnp.int32),
            pltpu.VMEM((CHUNK,), jnp.int32),
            pltpu.VMEM((CHUNK, OUT), jnp.float32),
            pltpu.VMEM((CHUNK, OUT), jnp.float32),
            pltpu.VMEM((CHUNK, OUT), jnp.float32),
            pltpu.VMEM((CHUNK, OUT), jnp.float32),
            pltpu.VMEM((N_DST,), jnp.float32),
            pltpu.VMEM_SHARED((N_DST, OUT), jnp.float32),
            pltpu.SemaphoreType.DMA,
            pltpu.SemaphoreType.DMA,
            pltpu.SemaphoreType.DMA,
            pltpu.SemaphoreType.DMA,
        ],
    )
    partial, dens = edge_call(edge_index_sa.reshape(2 * E_TOTAL),
                              asrc2.reshape(N_SRC), adst2.reshape(N_DST), hs)

    out = pl.pallas_call(
        _tail_body,
        out_shape=jax.ShapeDtypeStruct((N_DST, 1), jnp.float32),
    )(partial, dens, W1, b1.reshape(1, HID), W2, b2.reshape(1, 1))
    return out


# speedup vs baseline: 1.3632x; 1.0062x over previous
import jax
import jax.numpy as jnp
from jax import lax
from jax.experimental import pallas as pl
from jax.experimental.pallas import tpu as pltpu
from jax.experimental.pallas import tpu_sc as plsc

NEG_SLOPE = 0.2
OUT = 16
HID = 64
N_DST = 10000
N_SRC = 10000
E_TOTAL = 320000

NC = 2
NS = 16
NW = NC * NS
LANES = 16

EPW = E_TOTAL // NW
CHUNK = 400
NCHUNK = EPW // CHUNK
GROUPS = CHUNK // LANES
ROWS_PT = 624
ROWS_TAIL = N_DST - NS * ROWS_PT


def _proj_body(xs_ref, ws_ref, bs_ref, atts_ref, xa_ref, wa_ref, ba_ref,
               attd_ref, hs_ref, asrc_ref, adst_ref):
    hs = jnp.dot(xs_ref[...], ws_ref[...],
                 preferred_element_type=jnp.float32) + bs_ref[...]
    hs_ref[...] = hs
    asrc_ref[...] = jnp.dot(hs, atts_ref[...],
                            preferred_element_type=jnp.float32)
    ha = jnp.dot(xa_ref[...], wa_ref[...],
                 preferred_element_type=jnp.float32) + ba_ref[...]
    adst_ref[...] = jnp.dot(ha, attd_ref[...],
                            preferred_element_type=jnp.float32)


def _edge_body(ei_ref, asrc_hbm, adst_hbm, hs_hbm, out_hbm, den_hbm,
               asrc_v, adst_v, src0_v, src1_v, dst0_v, dst1_v,
               hr0_v, hr1_v, msg0_v, msg1_v, den_v,
               accum_sh, semi_s, semi_d, semh, sems_m):
    cid = lax.axis_index("c")
    sid = lax.axis_index("s")
    wid = sid * NC + cid

    src_b = [src0_v, src1_v]
    dst_b = [dst0_v, dst1_v]
    hr_b = [hr0_v, hr1_v]
    msg_b = [msg0_v, msg1_v]

    pltpu.sync_copy(asrc_hbm, asrc_v)
    pltpu.sync_copy(adst_hbm, adst_v)

    zeros16 = jnp.zeros((LANES,), jnp.float32)

    def zrow(r, carry):
        msg0_v[r, pl.ds(0, LANES)] = zeros16
        return carry
    lax.fori_loop(0, CHUNK, zrow, 0)

    def zd(r, carry):
        den_v[pl.ds(r * LANES, LANES)] = zeros16
        return carry
    lax.fori_loop(0, N_DST // LANES, zd, 0, unroll=8)

    base_row = sid * ROWS_PT
    pltpu.sync_copy(msg0_v, accum_sh.at[pl.ds(base_row, CHUNK)])
    pltpu.sync_copy(msg0_v.at[pl.ds(0, ROWS_PT - CHUNK)],
                    accum_sh.at[pl.ds(base_row + CHUNK, ROWS_PT - CHUNK)])

    @pl.when(sid == NS - 1)
    def _init_tail():
        pltpu.sync_copy(msg0_v.at[pl.ds(0, ROWS_TAIL)],
                        accum_sh.at[pl.ds(NS * ROWS_PT, ROWS_TAIL)])
    plsc.subcore_barrier()

    iota = lax.iota(jnp.int32, LANES)

    def issue_idx(c, b):
        off = wid * EPW + c * CHUNK
        d1 = pltpu.async_copy(ei_ref.at[pl.ds(off, CHUNK)], src_b[b], semi_s)
        d2 = pltpu.async_copy(ei_ref.at[pl.ds(E_TOTAL + off, CHUNK)],
                              dst_b[b], semi_d)
        return (d1, d2)

    def issue_gather(b):
        return pltpu.async_copy(hs_hbm.at[src_b[b]], hr_b[b], semh)

    def compute_chunk(b):
        src_v, dst_v, hrows_v, msg_v = src_b[b], dst_b[b], hr_b[b], msg_b[b]

        def group_body(g, gcarry):
            s16 = src_v[pl.ds(g * LANES, LANES)]
            d16 = dst_v[pl.ds(g * LANES, LANES)]
            a = plsc.load_gather(asrc_v, [s16]) + plsc.load_gather(adst_v, [d16])
            w = jnp.exp(jnp.maximum(a, a * NEG_SLOPE))
            plsc.addupdate_scatter(den_v, [d16], w)
            base = g * LANES
            for e in range(LANES):
                msg_v[base + e, pl.ds(0, LANES)] = (
                    hrows_v[base + e, pl.ds(0, LANES)] * w[e])
            return gcarry
        lax.fori_loop(0, GROUPS, group_body, 0, unroll=2)

    def issue_scatter(b):
        d1 = pltpu.async_copy(msg_b[b], accum_sh.at[dst_b[b]], sems_m, add=True)
        return (d1,)

    idx_d = issue_idx(0, 0)
    idx_d[0].wait()
    idx_d[1].wait()
    gat_d = issue_gather(0)
    scat_d = None
    for c in range(NCHUNK):
        b = c % 2
        nb = 1 - b
        if scat_d is not None:
            scat_d[0].wait()
        if c + 1 < NCHUNK:
            idx_d = issue_idx(c + 1, nb)
        gat_d.wait()
        if c + 1 < NCHUNK:
            idx_d[0].wait()
            idx_d[1].wait()
            gat_d = issue_gather(nb)
        compute_chunk(b)
        scat_d = issue_scatter(b)
    scat_d[0].wait()

    pltpu.sync_copy(den_v, den_hbm.at[wid])

    plsc.subcore_barrier()
    pltpu.sync_copy(accum_sh.at[pl.ds(base_row, ROWS_PT)],
                    out_hbm.at[cid, pl.ds(base_row, ROWS_PT)])

    @pl.when(sid == NS - 1)
    def _copy_tail():
        pltpu.sync_copy(accum_sh.at[pl.ds(NS * ROWS_PT, ROWS_TAIL)],
                        out_hbm.at[cid, pl.ds(NS * ROWS_PT, ROWS_TAIL)])


def _tail_body(part_ref, den_ref, w1_ref, b1_ref, w2_ref, b2_ref, out_ref):
    num = part_ref[0] + part_ref[1]
    den = jnp.sum(den_ref[...], axis=0).reshape(N_DST, 1)
    outa = jnp.maximum(num / (den + 1e-16), 0.0)
    h1 = jnp.maximum(jnp.dot(outa, w1_ref[...],
                             preferred_element_type=jnp.float32) + b1_ref[...], 0.0)
    logits = jnp.dot(h1, w2_ref[...],
                     preferred_element_type=jnp.float32) + b2_ref[...]
    m = jnp.max(logits)
    e = jnp.exp(logits - m)
    out_ref[...] = e / jnp.sum(e)


@jax.jit
def kernel(x_a, x_s, edge_index_sa, edge_index_as, W_a, b_a, W_s, b_s,
           att_src_sa, att_dst_sa, att_src_as, att_dst_as, q, Wk, bk,
           W1, b1, W2, b2):
    del edge_index_as, att_src_as, att_dst_as, q, Wk, bk

    hs, asrc2, adst2 = pl.pallas_call(
        _proj_body,
        out_shape=[
            jax.ShapeDtypeStruct((N_SRC, OUT), jnp.float32),
            jax.ShapeDtypeStruct((N_SRC, 1), jnp.float32),
            jax.ShapeDtypeStruct((N_DST, 1), jnp.float32),
        ],
    )(x_s, W_s, b_s.reshape(1, OUT), att_src_sa.reshape(OUT, 1),
      x_a, W_a, b_a.reshape(1, OUT), att_dst_sa.reshape(OUT, 1))

    mesh = plsc.VectorSubcoreMesh(core_axis_name="c", subcore_axis_name="s",
                                  num_cores=NC, num_subcores=NS)
    edge_call = pl.kernel(
        _edge_body,
        out_type=[
            jax.ShapeDtypeStruct((NC, N_DST, OUT), jnp.float32),
            jax.ShapeDtypeStruct((NW, N_DST), jnp.float32),
        ],
        mesh=mesh,
        compiler_params=pltpu.CompilerParams(needs_layout_passes=False,
                                             use_tc_tiling_on_sc=False),
        scratch_types=[
            pltpu.VMEM((N_SRC,), jnp.float32),
            pltpu.VMEM((N_DST,), jnp.float32),
            pltpu.VMEM((CHUNK,), jnp.int32),
            pltpu.VMEM((CHUNK,), jnp.int32),
            pltpu.VMEM((CHUNK,), jnp.int32),
            pltpu.VMEM((CHUNK,), jnp.int32),
            pltpu.VMEM((CHUNK, OUT), jnp.float32),
            pltpu.VMEM((CHUNK, OUT), jnp.float32),
            pltpu.VMEM((CHUNK, OUT), jnp.float32),
            pltpu.VMEM((CHUNK, OUT), jnp.float32),
            pltpu.VMEM((N_DST,), jnp.float32),
            pltpu.VMEM_SHARED((N_DST, OUT), jnp.float32),
            pltpu.SemaphoreType.DMA,
            pltpu.SemaphoreType.DMA,
            pltpu.SemaphoreType.DMA,
            pltpu.SemaphoreType.DMA,
        ],
    )
    partial, dens = edge_call(edge_index_sa.reshape(2 * E_TOTAL),
                              asrc2.reshape(N_SRC), adst2.reshape(N_DST), hs)

    out = pl.pallas_call(
        _tail_body,
        out_shape=jax.ShapeDtypeStruct((N_DST, 1), jnp.float32),
    )(partial, dens, W1, b1.reshape(1, HID), W2, b2.reshape(1, 1))
    return out
